# baseline (speedup 1.0000x reference)
import functools

import jax
import jax.numpy as jnp
from jax.experimental import pallas as pl
from jax.experimental.pallas import tpu as pltpu


def _fused_convrnn_kernel(x_ref, w_ref, b_ref, m_ref, y_ref, slab_ref, *,
                          T, cin, hd, kh, kw, W, NF, OFF, splits):
    ph, pw = kh // 2, kw // 2
    drs = list(range(-ph, ph + 1))
    b1, b2 = kh * cin, kh * (cin + hd)

    slab_ref[...] = jnp.zeros_like(slab_ref)

    def put_bands(base, n, v, s, nw):
        for i, dr in enumerate(drs):
            o = OFF + s - dr * W
            slab_ref[base + i * n:base + (i + 1) * n, o:o + nw] = v

    for k in range(T + 1):
        if k < T:
            put_bands(0, cin, x_ref[k].astype(slab_ref.dtype), 0, NF)
        gs = []
        M = 2 * hd
        for s, nw in splits:
            cc = jnp.dot(w_ref[...], slab_ref[:, OFF + s:OFF + s + nw],
                         preferred_element_type=jnp.float32)
            acc = cc[pw * M:(pw + 1) * M]
            for dc in range(-pw, pw + 1):
                if dc == 0:
                    continue
                c = cc[(dc + pw) * M:(dc + pw + 1) * M]
                acc += (pltpu.roll(c, (-dc) % nw, axis=1)
                        * m_ref[dc + pw, :, s:s + nw])
            gs.append(jnp.tanh(acc + b_ref[...]))
        for (s, nw), g in zip(splits, gs):
            if k < T:
                put_bands(b1, hd, g[:hd].astype(slab_ref.dtype), s, nw)
            if k >= 1:
                y_ref[k - 1, :, s:s + nw] = g[hd:]
                put_bands(b2, hd, g[hd:].astype(slab_ref.dtype), s, nw)


def _gate_slices(wx, wh, b, hd):
    wxg = wx[..., 3 * hd:4 * hd]
    whg = wh[..., 3 * hd:4 * hd]
    bg = b[:, 3 * hd:4 * hd].reshape(hd)
    wx2 = wxg.transpose(1, 3, 0, 2).reshape(wx.shape[1], hd, -1)
    wh2 = whg.transpose(1, 3, 0, 2).reshape(wh.shape[1], hd, -1)
    return wx2, wh2, bg


def kernel(x, wx0, wh0, b0, wx1, wh1, b1):
    T, B, cin, H, W = x.shape
    hd = wx0.shape[-1] // 4
    kh, kw = wx0.shape[0], wx0.shape[1]
    ph, pw = kh // 2, kw // 2
    NF = H * W
    OFF = 128
    EXT = OFF + NF + OFF
    K = kh * (cin + 2 * hd)

    splits = (((0, NF // 2), (NF // 2, NF // 2))
              if (NF % 256 == 0 and (NF // 2) % W == 0) else ((0, NF),))

    wx2_0, wh2_0, bg0 = _gate_slices(wx0, wh0, b0, hd)
    wx2_1, wh2_1, bg1 = _gate_slices(wx1, wh1, b1, hd)
    z_xh = jnp.zeros((kw, hd, kh * cin), jnp.float32)
    z_hh = jnp.zeros((kw, hd, kh * hd), jnp.float32)
    w_top = jnp.concatenate([wx2_0, wh2_0, z_hh], axis=2)
    w_bot = jnp.concatenate([z_xh, wx2_1, wh2_1], axis=2)
    w = jnp.concatenate([w_top, w_bot], axis=1).astype(jnp.bfloat16)
    w = w.reshape(kw * 2 * hd, K)
    bias = jnp.concatenate([bg0, bg1]).reshape(2 * hd, 1)

    col = jnp.arange(NF) % W
    shifts = jnp.arange(-pw, pw + 1).reshape(-1, 1)
    cm = ((col[None, :] + shifts >= 0)
          & (col[None, :] + shifts < W)).astype(jnp.float32)
    cm = jnp.broadcast_to(cm[:, None, :], (kw, 2 * hd, NF))

    xb = x.reshape(T, B, cin, NF)

    body = functools.partial(_fused_convrnn_kernel, T=T, cin=cin, hd=hd,
                             kh=kh, kw=kw, W=W, NF=NF, OFF=OFF,
                             splits=splits)

    y = pl.pallas_call(
        body,
        out_shape=jax.ShapeDtypeStruct((B, T, hd, NF), jnp.float32),
        grid=(B,),
        in_specs=[
            pl.BlockSpec((T, None, cin, NF), lambda b: (0, b, 0, 0)),
            pl.BlockSpec((kw * 2 * hd, K), lambda b: (0, 0)),
            pl.BlockSpec((2 * hd, 1), lambda b: (0, 0)),
            pl.BlockSpec((kw, 2 * hd, NF), lambda b: (0, 0, 0)),
        ],
        out_specs=pl.BlockSpec((None, T, hd, NF), lambda b: (b, 0, 0, 0)),
        scratch_shapes=[
            pltpu.VMEM((K, EXT), jnp.bfloat16),
        ],
        compiler_params=pltpu.CompilerParams(
            dimension_semantics=("arbitrary",),
        ),
        name="fused_convrnn2",
    )(xb, w, bias, cm)

    return y.reshape(B, T, hd, H, W)


# speedup vs baseline: 1.0583x; 1.0583x over previous
import functools

import jax
import jax.numpy as jnp
from jax.experimental import pallas as pl
from jax.experimental.pallas import tpu as pltpu


def _fused_convrnn_kernel(x_ref, w_ref, b_ref, m_ref, y_ref, slab_ref, *,
                          T, cin, hd, kh, kw, W, NF, OFF, splits):
    ph, pw = kh // 2, kw // 2
    drs = list(range(-ph, ph + 1))
    b1, b2 = kh * cin, kh * (cin + hd)

    slab_ref[...] = jnp.zeros_like(slab_ref)

    def put_bands(base, n, v, s, nw):
        for i, dr in enumerate(drs):
            o = OFF + s - dr * W
            slab_ref[base + i * n:base + (i + 1) * n, o:o + nw] = v

    for k in range(T + 1):
        if k < T:
            put_bands(0, cin, x_ref[k].astype(slab_ref.dtype), 0, NF)
        gs = []
        M = 2 * hd
        for s, nw in splits:
            acc = jnp.dot(w_ref[pw * M:(pw + 1) * M],
                          slab_ref[:, OFF + s:OFF + s + nw],
                          preferred_element_type=jnp.float32)
            for dc in range(-pw, pw + 1):
                if dc == 0:
                    continue
                c = jnp.dot(w_ref[(dc + pw) * M:(dc + pw + 1) * M],
                            slab_ref[:, OFF + s:OFF + s + nw],
                            preferred_element_type=jnp.float32)
                acc += (pltpu.roll(c, (-dc) % nw, axis=1)
                        * m_ref[dc + pw, :, s:s + nw])
            gs.append(jnp.tanh(acc + b_ref[...]))
        for (s, nw), g in zip(splits, gs):
            if k < T:
                put_bands(b1, hd, g[:hd].astype(slab_ref.dtype), s, nw)
            if k >= 1:
                y_ref[k - 1, :, s:s + nw] = g[hd:]
                put_bands(b2, hd, g[hd:].astype(slab_ref.dtype), s, nw)


def _gate_slices(wx, wh, b, hd):
    wxg = wx[..., 3 * hd:4 * hd]
    whg = wh[..., 3 * hd:4 * hd]
    bg = b[:, 3 * hd:4 * hd].reshape(hd)
    wx2 = wxg.transpose(1, 3, 0, 2).reshape(wx.shape[1], hd, -1)
    wh2 = whg.transpose(1, 3, 0, 2).reshape(wh.shape[1], hd, -1)
    return wx2, wh2, bg


def kernel(x, wx0, wh0, b0, wx1, wh1, b1):
    T, B, cin, H, W = x.shape
    hd = wx0.shape[-1] // 4
    kh, kw = wx0.shape[0], wx0.shape[1]
    ph, pw = kh // 2, kw // 2
    NF = H * W
    OFF = 128
    EXT = OFF + NF + OFF
    K = kh * (cin + 2 * hd)

    splits = (((0, NF // 2), (NF // 2, NF // 2))
              if (NF % 256 == 0 and (NF // 2) % W == 0) else ((0, NF),))

    wx2_0, wh2_0, bg0 = _gate_slices(wx0, wh0, b0, hd)
    wx2_1, wh2_1, bg1 = _gate_slices(wx1, wh1, b1, hd)
    z_xh = jnp.zeros((kw, hd, kh * cin), jnp.float32)
    z_hh = jnp.zeros((kw, hd, kh * hd), jnp.float32)
    w_top = jnp.concatenate([wx2_0, wh2_0, z_hh], axis=2)
    w_bot = jnp.concatenate([z_xh, wx2_1, wh2_1], axis=2)
    w = jnp.concatenate([w_top, w_bot], axis=1).astype(jnp.bfloat16)
    w = w.reshape(kw * 2 * hd, K)
    bias = jnp.concatenate([bg0, bg1]).reshape(2 * hd, 1)

    col = jnp.arange(NF) % W
    shifts = jnp.arange(-pw, pw + 1).reshape(-1, 1)
    cm = ((col[None, :] + shifts >= 0)
          & (col[None, :] + shifts < W)).astype(jnp.float32)
    cm = jnp.broadcast_to(cm[:, None, :], (kw, 2 * hd, NF))

    xb = x.reshape(T, B, cin, NF)

    body = functools.partial(_fused_convrnn_kernel, T=T, cin=cin, hd=hd,
                             kh=kh, kw=kw, W=W, NF=NF, OFF=OFF,
                             splits=splits)

    y = pl.pallas_call(
        body,
        out_shape=jax.ShapeDtypeStruct((B, T, hd, NF), jnp.float32),
        grid=(B,),
        in_specs=[
            pl.BlockSpec((T, None, cin, NF), lambda b: (0, b, 0, 0)),
            pl.BlockSpec((kw * 2 * hd, K), lambda b: (0, 0)),
            pl.BlockSpec((2 * hd, 1), lambda b: (0, 0)),
            pl.BlockSpec((kw, 2 * hd, NF), lambda b: (0, 0, 0)),
        ],
        out_specs=pl.BlockSpec((None, T, hd, NF), lambda b: (b, 0, 0, 0)),
        scratch_shapes=[
            pltpu.VMEM((K, EXT), jnp.bfloat16),
        ],
        compiler_params=pltpu.CompilerParams(
            dimension_semantics=("arbitrary",),
        ),
        name="fused_convrnn2",
    )(xb, w, bias, cm)

    return y.reshape(B, T, hd, H, W)


# speedup vs baseline: 1.2491x; 1.1803x over previous
import functools

import jax
import jax.numpy as jnp
from jax.experimental import pallas as pl
from jax.experimental.pallas import tpu as pltpu


def _fused_convrnn_kernel(x_ref, w_ref, b_ref, m_ref, y_ref, slab_ref, *,
                          T, cin, hd, kh, kw, W, NF, OFF, splits):
    ph, pw = kh // 2, kw // 2
    drs = list(range(-ph, ph + 1))
    b1, b2 = kh * cin, kh * (cin + hd)
    NB = slab_ref.shape[0]

    slab_ref[...] = jnp.zeros_like(slab_ref)

    def put_bands(j, base, n, v, s, nw):
        for i, dr in enumerate(drs):
            o = OFF + s - dr * W
            slab_ref[j, base + i * n:base + (i + 1) * n, o:o + nw] = v

    M = 2 * hd
    for k in range(T + 1):
        if k < T:
            for j in range(NB):
                put_bands(j, 0, cin, x_ref[k, j].astype(slab_ref.dtype),
                          0, NF)
        gs = []
        for j in range(NB):
            for s, nw in splits:
                acc = jnp.dot(w_ref[pw * M:(pw + 1) * M],
                              slab_ref[j, :, OFF + s:OFF + s + nw],
                              preferred_element_type=jnp.float32)
                for dc in range(-pw, pw + 1):
                    if dc == 0:
                        continue
                    c = jnp.dot(w_ref[(dc + pw) * M:(dc + pw + 1) * M],
                                slab_ref[j, :, OFF + s:OFF + s + nw],
                                preferred_element_type=jnp.float32)
                    acc += (pltpu.roll(c, (-dc) % nw, axis=1)
                            * m_ref[dc + pw, :, s:s + nw])
                gs.append((j, s, nw, jnp.tanh(acc + b_ref[...])))
        for j, s, nw, g in gs:
            if k < T:
                put_bands(j, b1, hd, g[:hd].astype(slab_ref.dtype), s, nw)
            if k >= 1:
                y_ref[j, k - 1, :, s:s + nw] = g[hd:]
                put_bands(j, b2, hd, g[hd:].astype(slab_ref.dtype), s, nw)


def _gate_slices(wx, wh, b, hd):
    wxg = wx[..., 3 * hd:4 * hd]
    whg = wh[..., 3 * hd:4 * hd]
    bg = b[:, 3 * hd:4 * hd].reshape(hd)
    wx2 = wxg.transpose(1, 3, 0, 2).reshape(wx.shape[1], hd, -1)
    wh2 = whg.transpose(1, 3, 0, 2).reshape(wh.shape[1], hd, -1)
    return wx2, wh2, bg


def kernel(x, wx0, wh0, b0, wx1, wh1, b1):
    T, B, cin, H, W = x.shape
    hd = wx0.shape[-1] // 4
    kh, kw = wx0.shape[0], wx0.shape[1]
    ph, pw = kh // 2, kw // 2
    NF = H * W
    OFF = 128
    EXT = OFF + NF + OFF
    K = kh * (cin + 2 * hd)

    splits = (((0, NF // 2), (NF // 2, NF // 2))
              if (NF % 256 == 0 and (NF // 2) % W == 0) else ((0, NF),))

    wx2_0, wh2_0, bg0 = _gate_slices(wx0, wh0, b0, hd)
    wx2_1, wh2_1, bg1 = _gate_slices(wx1, wh1, b1, hd)
    z_xh = jnp.zeros((kw, hd, kh * cin), jnp.float32)
    z_hh = jnp.zeros((kw, hd, kh * hd), jnp.float32)
    w_top = jnp.concatenate([wx2_0, wh2_0, z_hh], axis=2)
    w_bot = jnp.concatenate([z_xh, wx2_1, wh2_1], axis=2)
    w = jnp.concatenate([w_top, w_bot], axis=1).astype(jnp.bfloat16)
    w = w.reshape(kw * 2 * hd, K)
    bias = jnp.concatenate([bg0, bg1]).reshape(2 * hd, 1)

    col = jnp.arange(NF) % W
    shifts = jnp.arange(-pw, pw + 1).reshape(-1, 1)
    cm = ((col[None, :] + shifts >= 0)
          & (col[None, :] + shifts < W)).astype(jnp.float32)
    cm = jnp.broadcast_to(cm[:, None, :], (kw, 2 * hd, NF))

    NB = 2 if B % 2 == 0 else 1
    xb = x.reshape(T, B, cin, NF)

    body = functools.partial(_fused_convrnn_kernel, T=T, cin=cin, hd=hd,
                             kh=kh, kw=kw, W=W, NF=NF, OFF=OFF,
                             splits=splits)

    y = pl.pallas_call(
        body,
        out_shape=jax.ShapeDtypeStruct((B, T, hd, NF), jnp.float32),
        grid=(B // NB,),
        in_specs=[
            pl.BlockSpec((T, NB, cin, NF), lambda b: (0, b, 0, 0)),
            pl.BlockSpec((kw * 2 * hd, K), lambda b: (0, 0)),
            pl.BlockSpec((2 * hd, 1), lambda b: (0, 0)),
            pl.BlockSpec((kw, 2 * hd, NF), lambda b: (0, 0, 0)),
        ],
        out_specs=pl.BlockSpec((NB, T, hd, NF), lambda b: (b, 0, 0, 0)),
        scratch_shapes=[
            pltpu.VMEM((NB, K, EXT), jnp.bfloat16),
        ],
        compiler_params=pltpu.CompilerParams(
            dimension_semantics=("arbitrary",),
            vmem_limit_bytes=56 * 1024 * 1024,
        ),
        name="fused_convrnn2",
    )(xb, w, bias, cm)

    return y.reshape(B, T, hd, H, W)
